# Initial kernel scaffold; baseline (speedup 1.0000x reference)
import jax
import jax.numpy as jnp
from jax.experimental import pallas as pl


def kernel(xyz):
    raise NotImplementedError("write your pallas kernel here")


# speedup vs baseline: 15.6530x; 15.6530x over previous
import functools

import jax
import jax.numpy as jnp
import numpy as np
from jax.experimental import pallas as pl

B = 4
N = 4096
K = 16
NUM_FPS = 256
QBLK = 128

RECIP15 = float(np.float32(1.0) / np.float32(15.0))
RECIP16 = 0.0625
BIGI = np.int32(2**30)


def _fold(vals):
    while len(vals) > 1:
        h = len(vals) // 2
        vals = [vals[i] + vals[i + h] for i in range(h)]
    return vals[0]


def _bf16(x):
    return x.astype(jnp.bfloat16).astype(jnp.float32)


def _rot(c, s, u, v):
    return c * u - s * v, s * u + c * v


_BL4 = ((0, 2), (1, 3), (0, 3), (2, 1), (0, 1), (3, 2))


def _jacobi_v0(cov, sweeps=8):
    z = jnp.zeros_like(cov[(0, 0)])
    M = [[None] * 4 for _ in range(4)]
    for i in range(4):
        for j in range(4):
            if i < 3 and j < 3:
                M[i][j] = cov[(min(i, j), max(i, j))]
            else:
                M[i][j] = z
    one = jnp.ones_like(z)
    V = [[one if i == j else z for j in range(4)] for i in range(4)]
    for _ in range(sweeps):
        for (p, q) in _BL4:
            app, aqq, apq = M[p][p], M[q][q], M[p][q]
            tau = (aqq - app) / (2.0 * apq)
            sg = jnp.where(tau >= 0, 1.0, -1.0).astype(jnp.float32)
            t = sg / (jnp.abs(tau) + jnp.sqrt(1.0 + tau * tau))
            t = jnp.where(jnp.isnan(t), 0.0, t)
            t = jnp.where(apq == 0.0, 0.0, t)
            c = 1.0 / jnp.sqrt(1.0 + t * t)
            s = c * t
            for j in range(4):
                M[p][j], M[q][j] = _rot(c, s, M[p][j], M[q][j])
            for i in range(4):
                M[i][p], M[i][q] = _rot(c, s, M[i][p], M[i][q])
            for i in range(4):
                V[i][p], V[i][q] = _rot(c, s, V[i][p], V[i][q])
    ev = [M[0][0], M[1][1], M[2][2]]
    best = ev[0]
    i0 = jnp.zeros_like(best, dtype=jnp.int32)
    for i in (1, 2):
        take = ev[i] < best
        best = jnp.where(take, ev[i], best)
        i0 = jnp.where(take, np.int32(i), i0)
    v0 = []
    for r in range(3):
        x = jnp.where(i0 == 0, V[r][0], jnp.where(i0 == 1, V[r][1], V[r][2]))
        v0.append(x)
    return v0


def _normalize3(x, y, z):
    n2 = (x * x + y * y) + z * z
    n = jnp.maximum(jnp.sqrt(n2), 1e-12)
    return x / n, y / n, z / n


def _knn_normals_kernel(psub_ref, plane_ref, idx_ref, nrm_ref, lm_ref):
    px = psub_ref[0, :, 0:1]
    py = psub_ref[0, :, 1:2]
    pz = psub_ref[0, :, 2:3]
    qx = plane_ref[0, 0:1, :]
    qy = plane_ref[0, 1:2, :]
    qz = plane_ref[0, 2:3, :]
    dx = px - qx
    dy = py - qy
    dz = pz - qz
    d = (dx * dx + dy * dy) + dz * dz
    iota = jax.lax.broadcasted_iota(jnp.int32, (N, QBLK), 0)
    nbx, nby, nbz = [], [], []
    for k in range(K):
        m = jnp.min(d, axis=0, keepdims=True)
        cand = jnp.where(d == m, iota, BIGI)
        ik = jnp.min(cand, axis=0, keepdims=True)
        selm = iota == ik
        nbx.append(jnp.sum(jnp.where(selm, px, 0.0), axis=0, keepdims=True))
        nby.append(jnp.sum(jnp.where(selm, py, 0.0), axis=0, keepdims=True))
        nbz.append(jnp.sum(jnp.where(selm, pz, 0.0), axis=0, keepdims=True))
        d = jnp.where(selm, jnp.inf, d)
        idx_ref[0, k : k + 1, :] = ik
    mx = _fold(list(nbx)) * RECIP16
    my = _fold(list(nby)) * RECIP16
    mz = _fold(list(nbz)) * RECIP16
    cxs = [_bf16(v - mx) for v in nbx]
    cys = [_bf16(v - my) for v in nby]
    czs = [_bf16(v - mz) for v in nbz]
    cov = {}
    for (a, b), (la, lb) in {
        (0, 0): (cxs, cxs), (0, 1): (cxs, cys), (0, 2): (cxs, czs),
        (1, 1): (cys, cys), (1, 2): (cys, czs), (2, 2): (czs, czs),
    }.items():
        cov[(a, b)] = _fold([u * v for u, v in zip(la, lb)]) * RECIP15
    v0 = _jacobi_v0(cov)
    nx, ny, nz = _normalize3(*v0)
    nrm_ref[0, 0:1, :] = nx
    nrm_ref[0, 1:2, :] = ny
    nrm_ref[0, 2:3, :] = nz
    lm_ref[0, 0:1, :] = mx
    lm_ref[0, 1:2, :] = my
    lm_ref[0, 2:3, :] = mz


def _gather_mean_kernel(nsub_ref, idx_ref, nnm_ref):
    nx = nsub_ref[0, :, 0:1]
    ny = nsub_ref[0, :, 1:2]
    nz = nsub_ref[0, :, 2:3]
    iota = jax.lax.broadcasted_iota(jnp.int32, (N, QBLK), 0)
    gx, gy, gz = [], [], []
    for k in range(K):
        ik = idx_ref[0, k : k + 1, :]
        selm = iota == ik
        gx.append(jnp.sum(jnp.where(selm, nx, 0.0), axis=0, keepdims=True))
        gy.append(jnp.sum(jnp.where(selm, ny, 0.0), axis=0, keepdims=True))
        gz.append(jnp.sum(jnp.where(selm, nz, 0.0), axis=0, keepdims=True))
    nnm_ref[0, 0:1, :] = _fold(gx) * RECIP16
    nnm_ref[0, 1:2, :] = _fold(gy) * RECIP16
    nnm_ref[0, 2:3, :] = _fold(gz) * RECIP16


def _xnu_kernel(xyzt_ref, lm_ref, nnm_ref, mn_ref, xnu_ref):
    xx = xyzt_ref[0, 0:1, :]
    xy = xyzt_ref[0, 1:2, :]
    xz = xyzt_ref[0, 2:3, :]
    mnx, mny, mnz = _normalize3(
        nnm_ref[0, 0:1, :], nnm_ref[0, 1:2, :], nnm_ref[0, 2:3, :]
    )
    dx = xx - lm_ref[0, 0:1, :]
    dy = xy - lm_ref[0, 1:2, :]
    dz = xz - lm_ref[0, 2:3, :]
    p = [[_bf16(a * b) for b in (mnx, mny, mnz)] for a in (mnx, mny, mnz)]
    bx, by, bz = _bf16(dx), _bf16(dy), _bf16(dz)
    dnx = (p[0][0] * bx + p[0][1] * by) + p[0][2] * bz
    dny = (p[1][0] * bx + p[1][1] * by) + p[1][2] * bz
    dnz = (p[2][0] * bx + p[2][1] * by) + p[2][2] * bz
    xnu_ref[0, 0:1, :] = xx - dnx
    xnu_ref[0, 1:2, :] = xy - dny
    xnu_ref[0, 2:3, :] = xz - dnz
    mn_ref[0, 0:1, :] = mnx
    mn_ref[0, 1:2, :] = mny
    mn_ref[0, 2:3, :] = mnz


def _fps_kernel(xnut_ref, fps_ref):
    x = xnut_ref[:, 0, :]
    y = xnut_ref[:, 1, :]
    z = xnut_ref[:, 2, :]
    iota = jax.lax.broadcasted_iota(jnp.int32, (B, N), 1)
    iota_f = jax.lax.broadcasted_iota(jnp.int32, (B, NUM_FPS), 1)
    lx = x[:, 0:1]
    ly = y[:, 0:1]
    lz = z[:, 0:1]
    dx = x - lx
    dy = y - ly
    dz = z - lz
    dist = (dx * dx + dy * dy) + dz * dz
    fx = jnp.where(iota_f == 0, lx, 0.0)
    fy = jnp.where(iota_f == 0, ly, 0.0)
    fz = jnp.where(iota_f == 0, lz, 0.0)

    def body(s, carry):
        dist, fx, fy, fz = carry
        m = jnp.max(dist, axis=1, keepdims=True)
        cand = jnp.where(dist == m, iota, BIGI)
        ik = jnp.min(cand, axis=1, keepdims=True)
        selm = iota == ik
        lx = jnp.sum(jnp.where(selm, x, 0.0), axis=1, keepdims=True)
        ly = jnp.sum(jnp.where(selm, y, 0.0), axis=1, keepdims=True)
        lz = jnp.sum(jnp.where(selm, z, 0.0), axis=1, keepdims=True)
        dx = x - lx
        dy = y - ly
        dz = z - lz
        nd = (dx * dx + dy * dy) + dz * dz
        dist = jnp.minimum(dist, nd)
        fx = jnp.where(iota_f == s, lx, fx)
        fy = jnp.where(iota_f == s, ly, fy)
        fz = jnp.where(iota_f == s, lz, fz)
        return dist, fx, fy, fz

    _, fx, fy, fz = jax.lax.fori_loop(1, NUM_FPS, body, (dist, fx, fy, fz))
    fps_ref[:, 0, :] = fx
    fps_ref[:, 1, :] = fy
    fps_ref[:, 2, :] = fz


def _knn2_finish_kernel(xsub_ref, fps_ref, mnsub_ref, out_ref):
    px = xsub_ref[0, :, 0:1]
    py = xsub_ref[0, :, 1:2]
    pz = xsub_ref[0, :, 2:3]
    mx_ = mnsub_ref[0, :, 0:1]
    my_ = mnsub_ref[0, :, 1:2]
    mz_ = mnsub_ref[0, :, 2:3]
    qx = fps_ref[0, 0:1, :]
    qy = fps_ref[0, 1:2, :]
    qz = fps_ref[0, 2:3, :]
    dx = px - qx
    dy = py - qy
    dz = pz - qz
    d = (dx * dx + dy * dy) + dz * dz
    iota = jax.lax.broadcasted_iota(jnp.int32, (N, QBLK), 0)
    nbx, nby, nbz, gnx, gny, gnz = [], [], [], [], [], []
    for k in range(K):
        m = jnp.min(d, axis=0, keepdims=True)
        cand = jnp.where(d == m, iota, BIGI)
        ik = jnp.min(cand, axis=0, keepdims=True)
        selm = iota == ik
        nbx.append(jnp.sum(jnp.where(selm, px, 0.0), axis=0, keepdims=True))
        nby.append(jnp.sum(jnp.where(selm, py, 0.0), axis=0, keepdims=True))
        nbz.append(jnp.sum(jnp.where(selm, pz, 0.0), axis=0, keepdims=True))
        gnx.append(jnp.sum(jnp.where(selm, mx_, 0.0), axis=0, keepdims=True))
        gny.append(jnp.sum(jnp.where(selm, my_, 0.0), axis=0, keepdims=True))
        gnz.append(jnp.sum(jnp.where(selm, mz_, 0.0), axis=0, keepdims=True))
        d = jnp.where(selm, jnp.inf, d)
    lmx = _fold(nbx) * RECIP16
    lmy = _fold(nby) * RECIP16
    lmz = _fold(nbz) * RECIP16
    mnx, mny, mnz = _normalize3(
        _fold(gnx) * RECIP16, _fold(gny) * RECIP16, _fold(gnz) * RECIP16
    )
    dfx = qx - lmx
    dfy = qy - lmy
    dfz = qz - lmz
    mnv = (mnx, mny, mnz)
    p = [
        [
            _bf16((1.0 if i == j else 0.0) - mnv[i] * mnv[j])
            for j in range(3)
        ]
        for i in range(3)
    ]
    bx, by, bz = _bf16(dfx), _bf16(dfy), _bf16(dfz)
    dtx = (p[0][0] * bx + p[0][1] * by) + p[0][2] * bz
    dty = (p[1][0] * bx + p[1][1] * by) + p[1][2] * bz
    dtz = (p[2][0] * bx + p[2][1] * by) + p[2][2] * bz
    out_ref[0, 0:1, :] = qx - dtx
    out_ref[0, 1:2, :] = qy - dty
    out_ref[0, 2:3, :] = qz - dtz


@functools.partial(jax.jit, static_argnames=("interpret",))
def kernel(xyz, interpret=False):
    xyz = xyz.astype(jnp.float32)
    xyzt = jnp.transpose(xyz, (0, 2, 1))

    nq = N // QBLK
    idx, nrm, lm = pl.pallas_call(
        _knn_normals_kernel,
        grid=(B, nq),
        in_specs=[
            pl.BlockSpec((1, N, 3), lambda b, j: (b, 0, 0)),
            pl.BlockSpec((1, 3, QBLK), lambda b, j: (b, 0, j)),
        ],
        out_specs=[
            pl.BlockSpec((1, K, QBLK), lambda b, j: (b, 0, j)),
            pl.BlockSpec((1, 3, QBLK), lambda b, j: (b, 0, j)),
            pl.BlockSpec((1, 3, QBLK), lambda b, j: (b, 0, j)),
        ],
        out_shape=[
            jax.ShapeDtypeStruct((B, K, N), jnp.int32),
            jax.ShapeDtypeStruct((B, 3, N), jnp.float32),
            jax.ShapeDtypeStruct((B, 3, N), jnp.float32),
        ],
        interpret=interpret,
    )(xyz, xyzt)

    nrm_sub = jnp.transpose(nrm, (0, 2, 1))
    nnm = pl.pallas_call(
        _gather_mean_kernel,
        grid=(B, nq),
        in_specs=[
            pl.BlockSpec((1, N, 3), lambda b, j: (b, 0, 0)),
            pl.BlockSpec((1, K, QBLK), lambda b, j: (b, 0, j)),
        ],
        out_specs=pl.BlockSpec((1, 3, QBLK), lambda b, j: (b, 0, j)),
        out_shape=jax.ShapeDtypeStruct((B, 3, N), jnp.float32),
        interpret=interpret,
    )(nrm_sub, idx)

    mn, xnu = pl.pallas_call(
        _xnu_kernel,
        grid=(B,),
        in_specs=[
            pl.BlockSpec((1, 3, N), lambda b: (b, 0, 0)),
            pl.BlockSpec((1, 3, N), lambda b: (b, 0, 0)),
            pl.BlockSpec((1, 3, N), lambda b: (b, 0, 0)),
        ],
        out_specs=[
            pl.BlockSpec((1, 3, N), lambda b: (b, 0, 0)),
            pl.BlockSpec((1, 3, N), lambda b: (b, 0, 0)),
        ],
        out_shape=[
            jax.ShapeDtypeStruct((B, 3, N), jnp.float32),
            jax.ShapeDtypeStruct((B, 3, N), jnp.float32),
        ],
        interpret=interpret,
    )(xyzt, lm, nnm)

    fps = pl.pallas_call(
        _fps_kernel,
        grid=(1,),
        in_specs=[pl.BlockSpec((B, 3, N), lambda i: (0, 0, 0))],
        out_specs=pl.BlockSpec((B, 3, NUM_FPS), lambda i: (0, 0, 0)),
        out_shape=jax.ShapeDtypeStruct((B, 3, NUM_FPS), jnp.float32),
        interpret=interpret,
    )(xnu)

    xnu_sub = jnp.transpose(xnu, (0, 2, 1))
    mn_sub = jnp.transpose(mn, (0, 2, 1))
    out = pl.pallas_call(
        _knn2_finish_kernel,
        grid=(B, NUM_FPS // QBLK),
        in_specs=[
            pl.BlockSpec((1, N, 3), lambda b, j: (b, 0, 0)),
            pl.BlockSpec((1, 3, QBLK), lambda b, j: (b, 0, j)),
            pl.BlockSpec((1, N, 3), lambda b, j: (b, 0, 0)),
        ],
        out_specs=pl.BlockSpec((1, 3, QBLK), lambda b, j: (b, 0, j)),
        out_shape=jax.ShapeDtypeStruct((B, 3, NUM_FPS), jnp.float32),
        interpret=interpret,
    )(xnu_sub, fps, mn_sub)

    return jnp.transpose(out, (0, 2, 1))


# speedup vs baseline: 20.8288x; 1.3307x over previous
import functools

import jax
import jax.numpy as jnp
import numpy as np
from jax import lax
from jax.experimental import pallas as pl
from jax.experimental.pallas import tpu as pltpu
from jax.experimental.pallas import tpu_sc as plsc

B = 4
N = 4096
K = 16
NUM_FPS = 256
QBLK = 128

RECIP15 = float(np.float32(1.0) / np.float32(15.0))
RECIP16 = 0.0625
BIGI = np.int32(2**30)


def _fold(vals):
    while len(vals) > 1:
        h = len(vals) // 2
        vals = [vals[i] + vals[i + h] for i in range(h)]
    return vals[0]


def _adjtree(vals):
    while len(vals) > 1:
        vals = [vals[i] + vals[i + 1] for i in range(0, len(vals), 2)]
    return vals[0]


def _bf16(x):
    return x.astype(jnp.bfloat16).astype(jnp.float32)


def _rot(c, s, u, v):
    return c * u - s * v, s * u + c * v


_BL4 = ((0, 2), (1, 3), (0, 3), (2, 1), (0, 1), (3, 2))


def _jacobi_v0(cov, sweeps=8):
    z = jnp.zeros_like(cov[(0, 0)])
    M = [[None] * 4 for _ in range(4)]
    for i in range(4):
        for j in range(4):
            if i < 3 and j < 3:
                M[i][j] = cov[(min(i, j), max(i, j))]
            else:
                M[i][j] = z
    one = jnp.ones_like(z)
    V = [[one if i == j else z for j in range(4)] for i in range(4)]
    for _ in range(sweeps):
        for (p, q) in _BL4:
            app, aqq, apq = M[p][p], M[q][q], M[p][q]
            tau = (aqq - app) / (2.0 * apq)
            sg = jnp.where(tau >= 0, 1.0, -1.0).astype(jnp.float32)
            t = sg / (jnp.abs(tau) + jnp.sqrt(1.0 + tau * tau))
            t = jnp.where(jnp.isnan(t), 0.0, t)
            t = jnp.where(apq == 0.0, 0.0, t)
            c = 1.0 / jnp.sqrt(1.0 + t * t)
            s = c * t
            for j in range(4):
                M[p][j], M[q][j] = _rot(c, s, M[p][j], M[q][j])
            for i in range(4):
                M[i][p], M[i][q] = _rot(c, s, M[i][p], M[i][q])
            for i in range(4):
                V[i][p], V[i][q] = _rot(c, s, V[i][p], V[i][q])
    ev = [M[0][0], M[1][1], M[2][2]]
    best = ev[0]
    i0 = jnp.zeros_like(best, dtype=jnp.int32)
    for i in (1, 2):
        take = ev[i] < best
        best = jnp.where(take, ev[i], best)
        i0 = jnp.where(take, np.int32(i), i0)
    v0 = []
    for r in range(3):
        x = jnp.where(i0 == 0, V[r][0], jnp.where(i0 == 1, V[r][1], V[r][2]))
        v0.append(x)
    return v0


def _normalize3(x, y, z):
    n2 = (x * x + y * y) + z * z
    n = jnp.maximum(jnp.sqrt(n2), 1e-12)
    return x / n, y / n, z / n


def _knn_normals_kernel(psub_ref, plane_ref, idx_ref, nrm_ref, lm_ref):
    px = psub_ref[0, :, 0:1]
    py = psub_ref[0, :, 1:2]
    pz = psub_ref[0, :, 2:3]
    qx = plane_ref[0, 0:1, :]
    qy = plane_ref[0, 1:2, :]
    qz = plane_ref[0, 2:3, :]
    dx = px - qx
    dy = py - qy
    dz = pz - qz
    d = (dx * dx + dy * dy) + dz * dz
    iota = jax.lax.broadcasted_iota(jnp.int32, (N, QBLK), 0)
    nbx, nby, nbz = [], [], []
    for k in range(K):
        m = jnp.min(d, axis=0, keepdims=True)
        cand = jnp.where(d == m, iota, BIGI)
        ik = jnp.min(cand, axis=0, keepdims=True)
        selm = iota == ik
        nbx.append(jnp.sum(jnp.where(selm, px, 0.0), axis=0, keepdims=True))
        nby.append(jnp.sum(jnp.where(selm, py, 0.0), axis=0, keepdims=True))
        nbz.append(jnp.sum(jnp.where(selm, pz, 0.0), axis=0, keepdims=True))
        d = jnp.where(selm, jnp.inf, d)
        idx_ref[0, k : k + 1, :] = ik
    mx = _fold(list(nbx)) * RECIP16
    my = _fold(list(nby)) * RECIP16
    mz = _fold(list(nbz)) * RECIP16
    cxs = [_bf16(v - mx) for v in nbx]
    cys = [_bf16(v - my) for v in nby]
    czs = [_bf16(v - mz) for v in nbz]
    cov = {}
    for (a, b), (la, lb) in {
        (0, 0): (cxs, cxs), (0, 1): (cxs, cys), (0, 2): (cxs, czs),
        (1, 1): (cys, cys), (1, 2): (cys, czs), (2, 2): (czs, czs),
    }.items():
        cov[(a, b)] = _adjtree([u * v for u, v in zip(la, lb)]) * RECIP15
    v0 = _jacobi_v0(cov)
    nx, ny, nz = _normalize3(*v0)
    nrm_ref[0, 0:1, :] = nx
    nrm_ref[0, 1:2, :] = ny
    nrm_ref[0, 2:3, :] = nz
    lm_ref[0, 0:1, :] = mx
    lm_ref[0, 1:2, :] = my
    lm_ref[0, 2:3, :] = mz


_SC_TILES = 32
_SC_PTS = (B * N) // _SC_TILES
_SC_CHUNK = 128


def _sc_gather_mean(nrm_hbm, idx_hbm, out_hbm, table_v, idx_v, out_v):
    c = lax.axis_index("c")
    s = lax.axis_index("s")
    wid = s * 2 + c
    p0 = wid * _SC_PTS
    b = p0 // N
    i0 = p0 - b * N
    pltpu.sync_copy(nrm_hbm.at[b], table_v)
    for k in range(K):
        pltpu.sync_copy(
            idx_hbm.at[pl.ds((b * K + k) * N + i0, _SC_PTS)],
            idx_v.at[pl.ds(k * _SC_PTS, _SC_PTS)],
        )
    cidx = [jnp.full((16,), cc, jnp.int32) for cc in range(3)]
    for g in range(_SC_PTS // 16):
        ids = [idx_v[pl.ds(k * _SC_PTS + g * 16, 16)] for k in range(K)]
        for cc in range(3):
            regs = [plsc.load_gather(table_v, [ids[k], cidx[cc]]) for k in range(K)]
            while len(regs) > 1:
                h = len(regs) // 2
                regs = [regs[i] + regs[i + h] for i in range(h)]
            out_v[pl.ds(cc * _SC_PTS + g * 16, 16)] = regs[0] * 0.0625
    for cc in range(3):
        pltpu.sync_copy(
            out_v.at[pl.ds(cc * _SC_PTS, _SC_PTS)],
            out_hbm.at[pl.ds(cc * (B * N) + p0, _SC_PTS)],
        )


_sc_gather_mean_call = functools.partial(
    pl.kernel,
    out_type=jax.ShapeDtypeStruct((3 * B * N,), jnp.float32),
    mesh=plsc.VectorSubcoreMesh(core_axis_name="c", subcore_axis_name="s"),
    compiler_params=pltpu.CompilerParams(
        needs_layout_passes=False, use_tc_tiling_on_sc=False
    ),
    scratch_types=[
        pltpu.VMEM((N, 4), jnp.float32),
        pltpu.VMEM((K * _SC_PTS,), jnp.int32),
        pltpu.VMEM((3 * _SC_PTS,), jnp.float32),
    ],
)(_sc_gather_mean)


def _xnu_kernel(xyzt_ref, lm_ref, nnm_ref, mn_ref, xnu_ref):
    xx = xyzt_ref[0, 0:1, :]
    xy = xyzt_ref[0, 1:2, :]
    xz = xyzt_ref[0, 2:3, :]
    mnx, mny, mnz = _normalize3(
        nnm_ref[0, 0:1, :], nnm_ref[0, 1:2, :], nnm_ref[0, 2:3, :]
    )
    dx = xx - lm_ref[0, 0:1, :]
    dy = xy - lm_ref[0, 1:2, :]
    dz = xz - lm_ref[0, 2:3, :]
    p = [[_bf16(a * b) for b in (mnx, mny, mnz)] for a in (mnx, mny, mnz)]
    bx, by, bz = _bf16(dx), _bf16(dy), _bf16(dz)
    dnx = (p[0][0] * bx + p[0][1] * by) + p[0][2] * bz
    dny = (p[1][0] * bx + p[1][1] * by) + p[1][2] * bz
    dnz = (p[2][0] * bx + p[2][1] * by) + p[2][2] * bz
    xnu_ref[0, 0:1, :] = xx - dnx
    xnu_ref[0, 1:2, :] = xy - dny
    xnu_ref[0, 2:3, :] = xz - dnz
    mn_ref[0, 0:1, :] = mnx
    mn_ref[0, 1:2, :] = mny
    mn_ref[0, 2:3, :] = mnz


def _fps_kernel(xnut_ref, fps_ref):
    x = xnut_ref[:, 0, :]
    y = xnut_ref[:, 1, :]
    z = xnut_ref[:, 2, :]
    iota = jax.lax.broadcasted_iota(jnp.int32, (B, N), 1)
    iota_f = jax.lax.broadcasted_iota(jnp.int32, (B, NUM_FPS), 1)
    lx = x[:, 0:1]
    ly = y[:, 0:1]
    lz = z[:, 0:1]
    dx = x - lx
    dy = y - ly
    dz = z - lz
    dist = (dx * dx + dy * dy) + dz * dz
    fx = jnp.where(iota_f == 0, lx, 0.0)
    fy = jnp.where(iota_f == 0, ly, 0.0)
    fz = jnp.where(iota_f == 0, lz, 0.0)

    def body(s, carry):
        dist, fx, fy, fz = carry
        m = jnp.max(dist, axis=1, keepdims=True)
        cand = jnp.where(dist == m, iota, BIGI)
        ik = jnp.min(cand, axis=1, keepdims=True)
        selm = iota == ik
        lx = jnp.sum(jnp.where(selm, x, 0.0), axis=1, keepdims=True)
        ly = jnp.sum(jnp.where(selm, y, 0.0), axis=1, keepdims=True)
        lz = jnp.sum(jnp.where(selm, z, 0.0), axis=1, keepdims=True)
        dx = x - lx
        dy = y - ly
        dz = z - lz
        nd = (dx * dx + dy * dy) + dz * dz
        dist = jnp.minimum(dist, nd)
        fx = jnp.where(iota_f == s, lx, fx)
        fy = jnp.where(iota_f == s, ly, fy)
        fz = jnp.where(iota_f == s, lz, fz)
        return dist, fx, fy, fz

    _, fx, fy, fz = jax.lax.fori_loop(1, NUM_FPS, body, (dist, fx, fy, fz))
    fps_ref[:, 0, :] = fx
    fps_ref[:, 1, :] = fy
    fps_ref[:, 2, :] = fz


def _knn2_finish_kernel(xsub_ref, fps_ref, mnsub_ref, out_ref):
    px = xsub_ref[0, :, 0:1]
    py = xsub_ref[0, :, 1:2]
    pz = xsub_ref[0, :, 2:3]
    mx_ = mnsub_ref[0, :, 0:1]
    my_ = mnsub_ref[0, :, 1:2]
    mz_ = mnsub_ref[0, :, 2:3]
    qx = fps_ref[0, 0:1, :]
    qy = fps_ref[0, 1:2, :]
    qz = fps_ref[0, 2:3, :]
    dx = px - qx
    dy = py - qy
    dz = pz - qz
    d = (dx * dx + dy * dy) + dz * dz
    iota = jax.lax.broadcasted_iota(jnp.int32, (N, QBLK), 0)
    nbx, nby, nbz, gnx, gny, gnz = [], [], [], [], [], []
    for k in range(K):
        m = jnp.min(d, axis=0, keepdims=True)
        cand = jnp.where(d == m, iota, BIGI)
        ik = jnp.min(cand, axis=0, keepdims=True)
        selm = iota == ik
        nbx.append(jnp.sum(jnp.where(selm, px, 0.0), axis=0, keepdims=True))
        nby.append(jnp.sum(jnp.where(selm, py, 0.0), axis=0, keepdims=True))
        nbz.append(jnp.sum(jnp.where(selm, pz, 0.0), axis=0, keepdims=True))
        gnx.append(jnp.sum(jnp.where(selm, mx_, 0.0), axis=0, keepdims=True))
        gny.append(jnp.sum(jnp.where(selm, my_, 0.0), axis=0, keepdims=True))
        gnz.append(jnp.sum(jnp.where(selm, mz_, 0.0), axis=0, keepdims=True))
        d = jnp.where(selm, jnp.inf, d)
    lmx = _fold(nbx) * RECIP16
    lmy = _fold(nby) * RECIP16
    lmz = _fold(nbz) * RECIP16
    mnx, mny, mnz = _normalize3(
        _fold(gnx) * RECIP16, _fold(gny) * RECIP16, _fold(gnz) * RECIP16
    )
    dfx = qx - lmx
    dfy = qy - lmy
    dfz = qz - lmz
    mnv = (mnx, mny, mnz)
    p = [
        [
            _bf16((1.0 if i == j else 0.0) - mnv[i] * mnv[j])
            for j in range(3)
        ]
        for i in range(3)
    ]
    bx, by, bz = _bf16(dfx), _bf16(dfy), _bf16(dfz)
    dtx = (p[0][0] * bx + p[0][1] * by) + p[0][2] * bz
    dty = (p[1][0] * bx + p[1][1] * by) + p[1][2] * bz
    dtz = (p[2][0] * bx + p[2][1] * by) + p[2][2] * bz
    out_ref[0, 0:1, :] = qx - dtx
    out_ref[0, 1:2, :] = qy - dty
    out_ref[0, 2:3, :] = qz - dtz


@jax.jit
def kernel(xyz):
    xyz = xyz.astype(jnp.float32)
    xyzt = jnp.transpose(xyz, (0, 2, 1))

    nq = N // QBLK
    idx, nrm, lm = pl.pallas_call(
        _knn_normals_kernel,
        grid=(B, nq),
        in_specs=[
            pl.BlockSpec((1, N, 3), lambda b, j: (b, 0, 0)),
            pl.BlockSpec((1, 3, QBLK), lambda b, j: (b, 0, j)),
        ],
        out_specs=[
            pl.BlockSpec((1, K, QBLK), lambda b, j: (b, 0, j)),
            pl.BlockSpec((1, 3, QBLK), lambda b, j: (b, 0, j)),
            pl.BlockSpec((1, 3, QBLK), lambda b, j: (b, 0, j)),
        ],
        out_shape=[
            jax.ShapeDtypeStruct((B, K, N), jnp.int32),
            jax.ShapeDtypeStruct((B, 3, N), jnp.float32),
            jax.ShapeDtypeStruct((B, 3, N), jnp.float32),
        ],
    )(xyz, xyzt)

    nrm_sub = jnp.transpose(nrm, (0, 2, 1))
    nrm4 = jnp.pad(nrm_sub, ((0, 0), (0, 0), (0, 1)))
    idx_flat = jnp.reshape(idx, (B * K * N,))
    nnm_flat = _sc_gather_mean_call(nrm4, idx_flat)
    nnm = jnp.transpose(jnp.reshape(nnm_flat, (3, B, N)), (1, 0, 2))

    mn, xnu = pl.pallas_call(
        _xnu_kernel,
        grid=(B,),
        in_specs=[
            pl.BlockSpec((1, 3, N), lambda b: (b, 0, 0)),
            pl.BlockSpec((1, 3, N), lambda b: (b, 0, 0)),
            pl.BlockSpec((1, 3, N), lambda b: (b, 0, 0)),
        ],
        out_specs=[
            pl.BlockSpec((1, 3, N), lambda b: (b, 0, 0)),
            pl.BlockSpec((1, 3, N), lambda b: (b, 0, 0)),
        ],
        out_shape=[
            jax.ShapeDtypeStruct((B, 3, N), jnp.float32),
            jax.ShapeDtypeStruct((B, 3, N), jnp.float32),
        ],
    )(xyzt, lm, nnm)

    fps = pl.pallas_call(
        _fps_kernel,
        grid=(1,),
        in_specs=[pl.BlockSpec((B, 3, N), lambda i: (0, 0, 0))],
        out_specs=pl.BlockSpec((B, 3, NUM_FPS), lambda i: (0, 0, 0)),
        out_shape=jax.ShapeDtypeStruct((B, 3, NUM_FPS), jnp.float32),
    )(xnu)

    xnu_sub = jnp.transpose(xnu, (0, 2, 1))
    mn_sub = jnp.transpose(mn, (0, 2, 1))
    out = pl.pallas_call(
        _knn2_finish_kernel,
        grid=(B, NUM_FPS // QBLK),
        in_specs=[
            pl.BlockSpec((1, N, 3), lambda b, j: (b, 0, 0)),
            pl.BlockSpec((1, 3, QBLK), lambda b, j: (b, 0, j)),
            pl.BlockSpec((1, N, 3), lambda b, j: (b, 0, 0)),
        ],
        out_specs=pl.BlockSpec((1, 3, QBLK), lambda b, j: (b, 0, j)),
        out_shape=jax.ShapeDtypeStruct((B, 3, NUM_FPS), jnp.float32),
    )(xnu_sub, fps, mn_sub)

    return jnp.transpose(out, (0, 2, 1))
